# baseline (speedup 1.0000x reference)
import dataclasses

import jax
import jax.numpy as jnp
from jax.experimental import pallas as pl
from jax.experimental.pallas import tpu as pltpu
from jax.experimental.pallas import tpu_sc as plsc

N = 320000
R_IN = 128
R_OUT = 256
C_OUT = 256
NS = 10000

B = 4000
NB = N // B
KP = 136
NSP = 11264

SC_CORES = 2
SC_SUBCORES = 16
SC_WORKERS = SC_CORES * SC_SUBCORES
RPW = N // SC_WORKERS
HW = 512
NSC = 11264
NSCR = NSC // 8
STRIPE = NSCR // SC_SUBCORES


def _counts_kernel(bi_hbm, s0w_hbm, smaxw_hbm, out_hbm,
                   ids_v, hist_v, idxr_v, zbuf_v, bounds_v, shared):
    c = jax.lax.axis_index("c")
    s = jax.lax.axis_index("s")
    w = c * SC_SUBCORES + s

    lane = jax.lax.broadcasted_iota(jnp.int32, (16,), 0)
    zeros16 = jnp.zeros((16,), jnp.float32)
    ones16 = jnp.ones((16,), jnp.float32)

    @pl.loop(0, STRIPE)
    def _(k):
        @pl.loop(0, 128, step=16)
        def _(q):
            zbuf_v[k, pl.ds(q, 16)] = zeros16

    pltpu.sync_copy(zbuf_v, shared.at[pl.ds(s * STRIPE, STRIPE), :])

    pltpu.sync_copy(bi_hbm.at[pl.ds(w * RPW, RPW)], ids_v)
    pltpu.sync_copy(s0w_hbm, bounds_v.at[pl.ds(0, SC_WORKERS)])
    pltpu.sync_copy(smaxw_hbm, bounds_v.at[pl.ds(SC_WORKERS, SC_WORKERS)])
    neg = jnp.full((16,), -2147483648, jnp.int32)
    sel = lane == s
    s0v = jnp.where(sel, bounds_v[pl.ds(c * 16, 16)], neg)
    smv = jnp.where(sel, bounds_v[pl.ds(SC_WORKERS + c * 16, 16)], neg)
    base0 = (jnp.max(s0v) // 16) * 16
    smax = jnp.max(smv)
    nwin = (smax - base0) // HW + 1

    plsc.subcore_barrier()

    @pl.loop(0, nwin)
    def _(j):
        wb = base0 + j * HW

        @pl.loop(0, HW // 8)
        def _(k):
            @pl.loop(0, 128, step=16)
            def _(q):
                hist_v[k, pl.ds(q, 16)] = zeros16

        @pl.loop(0, RPW, step=16)
        def _(k):
            idv = ids_v[pl.ds(k, 16)]
            rel = idv - wb
            flat = (rel * 16) + lane
            rowi = flat // 128
            coli = flat - rowi * 128
            msk = (rel >= 0) & (rel < HW)
            plsc.addupdate_scatter(hist_v, [rowi, coli], ones16, mask=msk)

        @pl.loop(0, HW // 8, step=16)
        def _(k):
            idxr_v[0, pl.ds(k, 16)] = (wb // 8) + k + lane

        pltpu.sync_copy(hist_v, shared.at[idxr_v.at[0]], add=True)

    plsc.subcore_barrier()
    pltpu.sync_copy(shared.at[pl.ds(s * STRIPE, STRIPE), :],
                    out_hbm.at[c, pl.ds(s * STRIPE, STRIPE), :])


def _sc_counts(bi, s0w, smaxw):
    cp = pltpu.CompilerParams()
    if "needs_layout_passes" in pltpu.CompilerParams.__dataclass_fields__:
        cp = dataclasses.replace(cp, needs_layout_passes=False)
    kern = pl.kernel(
        _counts_kernel,
        compiler_params=cp,
        out_type=jax.ShapeDtypeStruct((SC_CORES, NSCR, 128), jnp.float32),
        mesh=plsc.VectorSubcoreMesh(core_axis_name="c", subcore_axis_name="s"),
        scratch_types=[
            pltpu.VMEM((RPW,), jnp.int32),
            pltpu.VMEM((HW // 8, 128), jnp.float32),
            pltpu.VMEM((1, HW // 8), jnp.int32),
            pltpu.VMEM((STRIPE, 128), jnp.float32),
            pltpu.VMEM((2 * SC_WORKERS,), jnp.int32),
            pltpu.VMEM_SHARED((NSCR, 128), jnp.float32),
        ],
    )
    return kern(bi, s0w, smaxw)


def _seg_kernel(s0_ref, smax_ref, ids_ref, x_ref, w1t_ref, b1_ref, sums_ref):
    i = pl.program_id(0)

    @pl.when(i == 0)
    def _():
        sums_ref[...] = jnp.zeros_like(sums_ref)

    xb = x_ref[...].astype(jnp.bfloat16)
    h = jnp.dot(xb, w1t_ref[...], preferred_element_type=jnp.float32)
    hb = jnp.maximum(h + b1_ref[...], 0).astype(jnp.bfloat16)

    ids = ids_ref[0]
    a0 = (s0_ref[i] // 8) * 8
    smax = smax_ref[i]
    nwin = (smax - a0) // KP + 1

    def win(j, carry):
        base = a0 + j * KP
        pos = ids - base
        row = jax.lax.broadcasted_iota(jnp.int32, (KP, B), 0)
        oh = row == pos
        ohf = oh.astype(jnp.bfloat16)
        ls = jax.lax.dot_general(ohf, hb, (((1,), (0,)), ((), ())),
                                 preferred_element_type=jnp.float32)
        sums_ref[pl.ds(base, KP), :] += ls
        return carry

    jax.lax.fori_loop(0, nwin, win, 0)


def _head_kernel(sums_ref, c0_ref, c1_ref, w2t_ref, b2_ref, out_ref):
    cr = c0_ref[0] + c1_ref[0]
    counts = jnp.sum(cr, axis=1, keepdims=True)
    mean = sums_ref[...] / jnp.maximum(counts, 1.0)
    out = jnp.dot(mean, w2t_ref[...], preferred_element_type=jnp.float32)
    out_ref[...] = jnp.maximum(out + b2_ref[...], 0.0)


def kernel(x, batch_index, W1, b1, W2, b2):
    bi = batch_index.astype(jnp.int32)
    s0 = bi[::B]
    smax = bi[B - 1::B]
    ids3 = bi.reshape(NB, 1, B)

    counts2 = _sc_counts(bi, bi[::RPW], bi[RPW - 1::RPW])
    counts2 = counts2.reshape(SC_CORES, NSP, 16)

    grid_spec = pltpu.PrefetchScalarGridSpec(
        num_scalar_prefetch=2,
        grid=(NB,),
        in_specs=[
            pl.BlockSpec((1, 1, B), lambda i, *_: (i, 0, 0)),
            pl.BlockSpec((B, R_IN), lambda i, *_: (i, 0)),
            pl.BlockSpec((R_IN, R_OUT), lambda i, *_: (0, 0)),
            pl.BlockSpec((1, R_OUT), lambda i, *_: (0, 0)),
        ],
        out_specs=[
            pl.BlockSpec((NSP, R_OUT), lambda i, *_: (0, 0)),
        ],
    )
    sums = pl.pallas_call(
        _seg_kernel,
        grid_spec=grid_spec,
        out_shape=[jax.ShapeDtypeStruct((NSP, R_OUT), jnp.float32)],
    )(s0, smax, ids3, x, W1.T.astype(jnp.bfloat16), b1.reshape(1, R_OUT))[0]

    R = 2816
    out = pl.pallas_call(
        _head_kernel,
        grid=(NSP // R,),
        in_specs=[
            pl.BlockSpec((R, R_OUT), lambda i: (i, 0)),
            pl.BlockSpec((1, R, 16), lambda i: (0, i, 0)),
            pl.BlockSpec((1, R, 16), lambda i: (1, i, 0)),
            pl.BlockSpec((R_OUT, C_OUT), lambda i: (0, 0)),
            pl.BlockSpec((1, C_OUT), lambda i: (0, 0)),
        ],
        out_specs=pl.BlockSpec((R, C_OUT), lambda i: (i, 0)),
        out_shape=jax.ShapeDtypeStruct((NSP, C_OUT), jnp.float32),
    )(sums, counts2, counts2, W2.T, b2.reshape(1, C_OUT))
    return out[:NS]


# speedup vs baseline: 1.0266x; 1.0266x over previous
import dataclasses

import jax
import jax.numpy as jnp
from jax.experimental import pallas as pl
from jax.experimental.pallas import tpu as pltpu
from jax.experimental.pallas import tpu_sc as plsc

N = 320000
R_IN = 128
R_OUT = 256
C_OUT = 256
NS = 10000

B = 4000
NB = N // B
KP = 136
NSP = 11264

SC_CORES = 1
SC_SUBCORES = 16
SC_WORKERS = SC_CORES * SC_SUBCORES
RPW = N // SC_WORKERS
HW = 512
NSC = 11264
NSCR = NSC // 8
STRIPE = NSCR // SC_SUBCORES


def _counts_kernel(bi_hbm, s0w_hbm, smaxw_hbm, out_hbm,
                   ids_v, hist_v, idxr_v, zbuf_v, bounds_v, shared):
    c = jax.lax.axis_index("c")
    s = jax.lax.axis_index("s")
    w = c * SC_SUBCORES + s

    lane = jax.lax.broadcasted_iota(jnp.int32, (16,), 0)
    zeros16 = jnp.zeros((16,), jnp.float32)
    ones16 = jnp.ones((16,), jnp.float32)

    @pl.loop(0, STRIPE)
    def _(k):
        @pl.loop(0, 128, step=16)
        def _(q):
            zbuf_v[k, pl.ds(q, 16)] = zeros16

    pltpu.sync_copy(zbuf_v, shared.at[pl.ds(s * STRIPE, STRIPE), :])

    pltpu.sync_copy(bi_hbm.at[pl.ds(w * RPW, RPW)], ids_v)
    pltpu.sync_copy(s0w_hbm, bounds_v.at[pl.ds(0, SC_WORKERS)])
    pltpu.sync_copy(smaxw_hbm, bounds_v.at[pl.ds(SC_WORKERS, SC_WORKERS)])
    neg = jnp.full((16,), -2147483648, jnp.int32)
    sel = lane == s
    s0v = jnp.where(sel, bounds_v[pl.ds(c * 16, 16)], neg)
    smv = jnp.where(sel, bounds_v[pl.ds(SC_WORKERS + c * 16, 16)], neg)
    base0 = (jnp.max(s0v) // 16) * 16
    smax = jnp.max(smv)
    nwin = (smax - base0) // HW + 1

    plsc.subcore_barrier()

    @pl.loop(0, nwin)
    def _(j):
        wb = base0 + j * HW

        @pl.loop(0, HW // 8)
        def _(k):
            @pl.loop(0, 128, step=16)
            def _(q):
                hist_v[k, pl.ds(q, 16)] = zeros16

        @pl.loop(0, RPW, step=16)
        def _(k):
            idv = ids_v[pl.ds(k, 16)]
            rel = idv - wb
            flat = (rel * 16) + lane
            rowi = flat // 128
            coli = flat - rowi * 128
            msk = (rel >= 0) & (rel < HW)
            plsc.addupdate_scatter(hist_v, [rowi, coli], ones16, mask=msk)

        @pl.loop(0, HW // 8, step=16)
        def _(k):
            idxr_v[0, pl.ds(k, 16)] = (wb // 8) + k + lane

        pltpu.sync_copy(hist_v, shared.at[idxr_v.at[0]], add=True)

    plsc.subcore_barrier()
    pltpu.sync_copy(shared.at[pl.ds(s * STRIPE, STRIPE), :],
                    out_hbm.at[c, pl.ds(s * STRIPE, STRIPE), :])


def _sc_counts(bi, s0w, smaxw):
    cp = pltpu.CompilerParams()
    if "needs_layout_passes" in pltpu.CompilerParams.__dataclass_fields__:
        cp = dataclasses.replace(cp, needs_layout_passes=False)
    kern = pl.kernel(
        _counts_kernel,
        compiler_params=cp,
        out_type=jax.ShapeDtypeStruct((SC_CORES, NSCR, 128), jnp.float32),
        mesh=plsc.VectorSubcoreMesh(core_axis_name="c", subcore_axis_name="s", num_cores=1),
        scratch_types=[
            pltpu.VMEM((RPW,), jnp.int32),
            pltpu.VMEM((HW // 8, 128), jnp.float32),
            pltpu.VMEM((1, HW // 8), jnp.int32),
            pltpu.VMEM((STRIPE, 128), jnp.float32),
            pltpu.VMEM((2 * SC_WORKERS,), jnp.int32),
            pltpu.VMEM_SHARED((NSCR, 128), jnp.float32),
        ],
    )
    return kern(bi, s0w, smaxw)


def _seg_kernel(s0_ref, smax_ref, ids_ref, x_ref, w1t_ref, b1_ref, sums_ref):
    i = pl.program_id(0)

    @pl.when(i == 0)
    def _():
        sums_ref[...] = jnp.zeros_like(sums_ref)

    xb = x_ref[...].astype(jnp.bfloat16)
    h = jnp.dot(xb, w1t_ref[...], preferred_element_type=jnp.float32)
    hb = jnp.maximum(h + b1_ref[...], 0).astype(jnp.bfloat16)

    ids = ids_ref[0]
    a0 = (s0_ref[i] // 8) * 8
    smax = smax_ref[i]
    nwin = (smax - a0) // KP + 1

    def win(j, carry):
        base = a0 + j * KP
        pos = ids - base
        row = jax.lax.broadcasted_iota(jnp.int32, (KP, B), 0)
        oh = row == pos
        ohf = oh.astype(jnp.bfloat16)
        ls = jax.lax.dot_general(ohf, hb, (((1,), (0,)), ((), ())),
                                 preferred_element_type=jnp.float32)
        sums_ref[pl.ds(base, KP), :] += ls
        return carry

    jax.lax.fori_loop(0, nwin, win, 0)


def _head_kernel(sums_ref, c0_ref, w2t_ref, b2_ref, out_ref):
    cr = c0_ref[0]
    counts = jnp.sum(cr, axis=1, keepdims=True)
    mean = sums_ref[...] / jnp.maximum(counts, 1.0)
    out = jnp.dot(mean, w2t_ref[...], preferred_element_type=jnp.float32)
    out_ref[...] = jnp.maximum(out + b2_ref[...], 0.0)


def kernel(x, batch_index, W1, b1, W2, b2):
    bi = batch_index.astype(jnp.int32)
    s0 = bi[::B]
    smax = bi[B - 1::B]
    ids3 = bi.reshape(NB, 1, B)

    counts2 = _sc_counts(bi, bi[::RPW], bi[RPW - 1::RPW])
    counts2 = counts2.reshape(SC_CORES, NSP, 16)

    grid_spec = pltpu.PrefetchScalarGridSpec(
        num_scalar_prefetch=2,
        grid=(NB,),
        in_specs=[
            pl.BlockSpec((1, 1, B), lambda i, *_: (i, 0, 0)),
            pl.BlockSpec((B, R_IN), lambda i, *_: (i, 0)),
            pl.BlockSpec((R_IN, R_OUT), lambda i, *_: (0, 0)),
            pl.BlockSpec((1, R_OUT), lambda i, *_: (0, 0)),
        ],
        out_specs=[
            pl.BlockSpec((NSP, R_OUT), lambda i, *_: (0, 0)),
        ],
    )
    sums = pl.pallas_call(
        _seg_kernel,
        grid_spec=grid_spec,
        out_shape=[jax.ShapeDtypeStruct((NSP, R_OUT), jnp.float32)],
    )(s0, smax, ids3, x, W1.T.astype(jnp.bfloat16), b1.reshape(1, R_OUT))[0]

    R = 2816
    out = pl.pallas_call(
        _head_kernel,
        grid=(NSP // R,),
        in_specs=[
            pl.BlockSpec((R, R_OUT), lambda i: (i, 0)),
            pl.BlockSpec((1, R, 16), lambda i: (0, i, 0)),
            pl.BlockSpec((R_OUT, C_OUT), lambda i: (0, 0)),
            pl.BlockSpec((1, C_OUT), lambda i: (0, 0)),
        ],
        out_specs=pl.BlockSpec((R, C_OUT), lambda i: (i, 0)),
        out_shape=jax.ShapeDtypeStruct((NSP, C_OUT), jnp.float32),
    )(sums, counts2, W2.T, b2.reshape(1, C_OUT))
    return out[:NS]


# speedup vs baseline: 1.0964x; 1.0680x over previous
import dataclasses

import jax
import jax.numpy as jnp
from jax.experimental import pallas as pl
from jax.experimental.pallas import tpu as pltpu
from jax.experimental.pallas import tpu_sc as plsc

N = 320000
R_IN = 128
R_OUT = 256
C_OUT = 256
NS = 10000

B = 6400
NB = N // B
KP = 216
NSP = 11264

SC_CORES = 1
SC_SUBCORES = 16
SC_WORKERS = SC_CORES * SC_SUBCORES
RPW = N // SC_WORKERS
HW = 512
NSC = 11264
NSCR = NSC // 8
STRIPE = NSCR // SC_SUBCORES


def _counts_kernel(bi_hbm, s0w_hbm, smaxw_hbm, out_hbm,
                   ids_v, hist_v, idxr_v, zbuf_v, bounds_v, shared):
    c = jax.lax.axis_index("c")
    s = jax.lax.axis_index("s")
    w = c * SC_SUBCORES + s

    lane = jax.lax.broadcasted_iota(jnp.int32, (16,), 0)
    zeros16 = jnp.zeros((16,), jnp.float32)
    ones16 = jnp.ones((16,), jnp.float32)

    @pl.loop(0, STRIPE)
    def _(k):
        @pl.loop(0, 128, step=16)
        def _(q):
            zbuf_v[k, pl.ds(q, 16)] = zeros16

    pltpu.sync_copy(zbuf_v, shared.at[pl.ds(s * STRIPE, STRIPE), :])

    pltpu.sync_copy(bi_hbm.at[pl.ds(w * RPW, RPW)], ids_v)
    pltpu.sync_copy(s0w_hbm, bounds_v.at[pl.ds(0, SC_WORKERS)])
    pltpu.sync_copy(smaxw_hbm, bounds_v.at[pl.ds(SC_WORKERS, SC_WORKERS)])
    neg = jnp.full((16,), -2147483648, jnp.int32)
    sel = lane == s
    s0v = jnp.where(sel, bounds_v[pl.ds(c * 16, 16)], neg)
    smv = jnp.where(sel, bounds_v[pl.ds(SC_WORKERS + c * 16, 16)], neg)
    base0 = (jnp.max(s0v) // 16) * 16
    smax = jnp.max(smv)
    nwin = (smax - base0) // HW + 1

    plsc.subcore_barrier()

    @pl.loop(0, nwin)
    def _(j):
        wb = base0 + j * HW

        @pl.loop(0, HW // 8)
        def _(k):
            @pl.loop(0, 128, step=16)
            def _(q):
                hist_v[k, pl.ds(q, 16)] = zeros16

        @pl.loop(0, RPW, step=16)
        def _(k):
            idv = ids_v[pl.ds(k, 16)]
            rel = idv - wb
            flat = (rel * 16) + lane
            rowi = flat // 128
            coli = flat - rowi * 128
            msk = (rel >= 0) & (rel < HW)
            plsc.addupdate_scatter(hist_v, [rowi, coli], ones16, mask=msk)

        @pl.loop(0, HW // 8, step=16)
        def _(k):
            idxr_v[0, pl.ds(k, 16)] = (wb // 8) + k + lane

        pltpu.sync_copy(hist_v, shared.at[idxr_v.at[0]], add=True)

    plsc.subcore_barrier()
    pltpu.sync_copy(shared.at[pl.ds(s * STRIPE, STRIPE), :],
                    out_hbm.at[c, pl.ds(s * STRIPE, STRIPE), :])


def _sc_counts(bi, s0w, smaxw):
    cp = pltpu.CompilerParams()
    if "needs_layout_passes" in pltpu.CompilerParams.__dataclass_fields__:
        cp = dataclasses.replace(cp, needs_layout_passes=False)
    kern = pl.kernel(
        _counts_kernel,
        compiler_params=cp,
        out_type=jax.ShapeDtypeStruct((SC_CORES, NSCR, 128), jnp.float32),
        mesh=plsc.VectorSubcoreMesh(core_axis_name="c", subcore_axis_name="s", num_cores=1),
        scratch_types=[
            pltpu.VMEM((RPW,), jnp.int32),
            pltpu.VMEM((HW // 8, 128), jnp.float32),
            pltpu.VMEM((1, HW // 8), jnp.int32),
            pltpu.VMEM((STRIPE, 128), jnp.float32),
            pltpu.VMEM((2 * SC_WORKERS,), jnp.int32),
            pltpu.VMEM_SHARED((NSCR, 128), jnp.float32),
        ],
    )
    return kern(bi, s0w, smaxw)


def _seg_kernel(s0_ref, smax_ref, ids_ref, x_ref, w1t_ref, b1_ref, sums_ref):
    i = pl.program_id(0)

    @pl.when(i == 0)
    def _():
        sums_ref[...] = jnp.zeros_like(sums_ref)

    xb = x_ref[...].astype(jnp.bfloat16)
    h = jnp.dot(xb, w1t_ref[...], preferred_element_type=jnp.float32)
    hb = jnp.maximum(h + b1_ref[...], 0).astype(jnp.bfloat16)

    ids = ids_ref[0]
    a0 = (s0_ref[i] // 8) * 8
    smax = smax_ref[i]
    nwin = (smax - a0) // KP + 1

    def win(j, carry):
        base = a0 + j * KP
        pos = ids - base
        row = jax.lax.broadcasted_iota(jnp.int32, (KP, B), 0)
        oh = row == pos
        ohf = oh.astype(jnp.bfloat16)
        ls = jax.lax.dot_general(ohf, hb, (((1,), (0,)), ((), ())),
                                 preferred_element_type=jnp.float32)
        sums_ref[pl.ds(base, KP), :] += ls
        return carry

    jax.lax.fori_loop(0, nwin, win, 0)


def _head_kernel(sums_ref, c0_ref, w2t_ref, b2_ref, out_ref):
    cr = c0_ref[0]
    counts = jnp.sum(cr, axis=1, keepdims=True)
    mean = sums_ref[...] / jnp.maximum(counts, 1.0)
    out = jnp.dot(mean, w2t_ref[...], preferred_element_type=jnp.float32)
    out_ref[...] = jnp.maximum(out + b2_ref[...], 0.0)


def kernel(x, batch_index, W1, b1, W2, b2):
    bi = batch_index.astype(jnp.int32)
    s0 = bi[::B]
    smax = bi[B - 1::B]
    ids3 = bi.reshape(NB, 1, B)

    counts2 = _sc_counts(bi, bi[::RPW], bi[RPW - 1::RPW])
    counts2 = counts2.reshape(SC_CORES, NSP, 16)

    grid_spec = pltpu.PrefetchScalarGridSpec(
        num_scalar_prefetch=2,
        grid=(NB,),
        in_specs=[
            pl.BlockSpec((1, 1, B), lambda i, *_: (i, 0, 0)),
            pl.BlockSpec((B, R_IN), lambda i, *_: (i, 0)),
            pl.BlockSpec((R_IN, R_OUT), lambda i, *_: (0, 0)),
            pl.BlockSpec((1, R_OUT), lambda i, *_: (0, 0)),
        ],
        out_specs=[
            pl.BlockSpec((NSP, R_OUT), lambda i, *_: (0, 0)),
        ],
    )
    sums = pl.pallas_call(
        _seg_kernel,
        grid_spec=grid_spec,
        out_shape=[jax.ShapeDtypeStruct((NSP, R_OUT), jnp.float32)],
    )(s0, smax, ids3, x, W1.T.astype(jnp.bfloat16), b1.reshape(1, R_OUT))[0]

    R = 2816
    out = pl.pallas_call(
        _head_kernel,
        grid=(NSP // R,),
        in_specs=[
            pl.BlockSpec((R, R_OUT), lambda i: (i, 0)),
            pl.BlockSpec((1, R, 16), lambda i: (0, i, 0)),
            pl.BlockSpec((R_OUT, C_OUT), lambda i: (0, 0)),
            pl.BlockSpec((1, C_OUT), lambda i: (0, 0)),
        ],
        out_specs=pl.BlockSpec((R, C_OUT), lambda i: (i, 0)),
        out_shape=jax.ShapeDtypeStruct((NSP, C_OUT), jnp.float32),
    )(sums, counts2, W2.T, b2.reshape(1, C_OUT))
    return out[:NS]
